# Initial kernel scaffold; baseline (speedup 1.0000x reference)
import jax
import jax.numpy as jnp
from jax.experimental import pallas as pl


def kernel(x, edge_index, diffusion_distance, W_init, b_init, W_final, b_final):
    raise NotImplementedError("write your pallas kernel here")


# speedup vs baseline: 1.2287x; 1.2287x over previous
import dataclasses

import jax
import jax.numpy as jnp
from jax import lax
from jax.experimental import pallas as pl
from jax.experimental.pallas import tpu as pltpu
from jax.experimental.pallas import tpu_sc as plsc

N = 10000
D = 128
E = 320000
NUM_LAYERS = 2
ALPHA = 0.1
BETA = 0.1
ETA = 0.5
EPS = 1e-5

NC = 2
NS = 16
NT = NC * NS
PER_TILE = E // NT
CH = 80
NCH = PER_TILE // CH
RPS = 624
TAIL = N - NS * RPS
RB = 5
BM = N // RB

_mesh = plsc.VectorSubcoreMesh(core_axis_name="c", subcore_axis_name="s")

_sc_params = pltpu.CompilerParams()
if "needs_layout_passes" in pltpu.CompilerParams.__dataclass_fields__:
    _sc_params = dataclasses.replace(_sc_params, needs_layout_passes=False)


def _deg_body(dst_hbm, out0_hbm, out1_hbm, deg_sh, dst_v, ones_v, zero_v):
    ci = lax.axis_index("c")
    si = lax.axis_index("s")
    tid = ci * NS + si

    @pl.loop(0, CH // 16)
    def _(i):
        ones_v[pl.ds(i * 16, 16)] = jnp.ones((16,), jnp.float32)

    @pl.when(si == 0)
    def _():
        @pl.loop(0, N // 16)
        def _(i):
            zero_v[pl.ds(i * 16, 16)] = jnp.zeros((16,), jnp.float32)
        pltpu.sync_copy(zero_v, deg_sh)

    pltpu.sync_copy(dst_hbm.at[tid], dst_v)
    plsc.subcore_barrier()

    @pl.loop(0, NCH)
    def _(c):
        pltpu.sync_copy(ones_v, deg_sh.at[dst_v.at[c]], add=True)

    plsc.subcore_barrier()

    @pl.when(jnp.logical_and(si == 0, ci == 0))
    def _():
        pltpu.sync_copy(deg_sh, out0_hbm)

    @pl.when(jnp.logical_and(si == 0, ci == 1))
    def _():
        pltpu.sync_copy(deg_sh, out1_hbm)


def _edge_body(h_hbm, w_hbm, src_hbm, dst_hbm, dd_hbm, out_hbm,
               agg_sh, w_v, sidx_v, didx_v, ddc_v, hs_v, hd_v, msg_v):
    ci = lax.axis_index("c")
    si = lax.axis_index("s")
    tid = ci * NS + si

    pltpu.sync_copy(w_hbm, w_v)

    @pl.loop(0, CH)
    def _(r):
        for cc in range(D // 16):
            msg_v[r, pl.ds(cc * 16, 16)] = jnp.zeros((16,), jnp.float32)

    for t in range(RPS // CH):
        pltpu.sync_copy(msg_v, agg_sh.at[pl.ds(si * RPS + t * CH, CH)])
    rem = RPS - (RPS // CH) * CH
    pltpu.sync_copy(msg_v.at[pl.ds(0, rem)],
                    agg_sh.at[pl.ds(si * RPS + RPS - rem, rem)])

    @pl.when(si == NS - 1)
    def _():
        pltpu.sync_copy(msg_v.at[pl.ds(0, TAIL)],
                        agg_sh.at[pl.ds(NS * RPS, TAIL)])

    plsc.subcore_barrier()

    iota16 = lax.iota(jnp.int32, 16)

    @pl.loop(0, NCH)
    def _chunk(c):
        pltpu.sync_copy(src_hbm.at[tid, c], sidx_v)
        pltpu.sync_copy(dst_hbm.at[tid, c], didx_v)
        pltpu.sync_copy(dd_hbm.at[tid, c], ddc_v)
        pltpu.sync_copy(h_hbm.at[sidx_v], hs_v)
        pltpu.sync_copy(h_hbm.at[didx_v], hd_v)
        for g in range(CH // 16):
            rb = g * 16
            src16 = sidx_v[pl.ds(rb, 16)]
            dst16 = didx_v[pl.ds(rb, 16)]
            dd16 = ddc_v[pl.ds(rb, 16)]
            ws = plsc.load_gather(w_v, [src16])
            wd = plsc.load_gather(w_v, [dst16])
            rows = rb + iota16

            def _acc(kk, acc):
                for j in range(16):
                    col = lax.broadcast(kk * 16 + j, (16,))
                    hs_k = plsc.load_gather(hs_v, [rows, col])
                    hd_k = plsc.load_gather(hd_v, [rows, col])
                    t = wd * hd_k - ws * hs_k
                    acc = acc + t * t
                return acc

            s = lax.fori_loop(0, D // 16, _acc, jnp.zeros((16,), jnp.float32))
            scl = jnp.maximum(s, 1e-20)
            ib = lax.bitcast_convert_type(scl, jnp.int32)
            ib = jnp.int32(0x5F3759DF) - lax.shift_right_arithmetic(ib, 1)
            y = lax.bitcast_convert_type(ib, jnp.float32)
            for _ in range(3):
                y = y * (1.5 - 0.5 * scl * y * y)
            denom = scl * y + EPS
            q = (ETA * dd16) / denom
            a16 = ws * wd * (1.0 - q)
            b16 = (ws * ws) * q

            @pl.loop(0, D // 16)
            def _msg(kk):
                for j in range(16):
                    col = lax.broadcast(kk * 16 + j, (16,))
                    hs_k = plsc.load_gather(hs_v, [rows, col])
                    hd_k = plsc.load_gather(hd_v, [rows, col])
                    plsc.store_scatter(msg_v, [rows, col],
                                       a16 * hs_k + b16 * hd_k)

        pltpu.sync_copy(msg_v, agg_sh.at[didx_v], add=True)

    plsc.subcore_barrier()
    pltpu.sync_copy(agg_sh.at[pl.ds(si * RPS, RPS)],
                    out_hbm.at[ci, pl.ds(si * RPS, RPS)])

    @pl.when(si == NS - 1)
    def _():
        pltpu.sync_copy(agg_sh.at[pl.ds(NS * RPS, TAIL)],
                        out_hbm.at[ci, pl.ds(NS * RPS, TAIL)])


def _deg_partials(dst_tiles):
    return pl.kernel(
        _deg_body,
        out_type=[jax.ShapeDtypeStruct((N,), jnp.float32),
                  jax.ShapeDtypeStruct((N,), jnp.float32)],
        mesh=_mesh,
        compiler_params=_sc_params,
        scratch_types=[
            pltpu.VMEM_SHARED((N,), jnp.float32),
            pltpu.VMEM((NCH, CH), jnp.int32),
            pltpu.VMEM((CH,), jnp.float32),
            pltpu.VMEM((N,), jnp.float32),
        ],
    )(dst_tiles)


def _edge_partials(h, w, src_tiles, dst_tiles, dd_tiles):
    return pl.kernel(
        _edge_body,
        out_type=jax.ShapeDtypeStruct((NC, N, D), jnp.float32),
        mesh=_mesh,
        compiler_params=_sc_params,
        scratch_types=[
            pltpu.VMEM_SHARED((N, D), jnp.float32),
            pltpu.VMEM((N,), jnp.float32),
            pltpu.VMEM((CH,), jnp.int32),
            pltpu.VMEM((CH,), jnp.int32),
            pltpu.VMEM((CH,), jnp.float32),
            pltpu.VMEM((CH, D), jnp.float32),
            pltpu.VMEM((CH, D), jnp.float32),
            pltpu.VMEM((CH, D), jnp.float32),
        ],
    )(h, w, src_tiles, dst_tiles, dd_tiles)


def _k1_body(x_ref, wi_ref, bi_ref, d0_ref, d1_ref, h_ref, w_ref):
    h_ref[...] = (jnp.dot(x_ref[...], wi_ref[...],
                          preferred_element_type=jnp.float32) + bi_ref[...])
    deg = d0_ref[...] + d1_ref[...]
    w_ref[...] = jnp.where(deg > 0.0, lax.rsqrt(jnp.maximum(deg, 1.0)), 0.0)


def _k2_body(h_ref, m_ref, acc_ref):
    i = pl.program_id(0)

    @pl.when(i == 0)
    def _():
        acc_ref[...] = jnp.zeros_like(acc_ref)

    hb = h_ref[...]
    acc_ref[...] += lax.dot_general(hb, hb, (((0,), (0,)), ((), ())),
                                    preferred_element_type=jnp.float32)

    @pl.when(i == RB - 1)
    def _():
        G = acc_ref[...]
        rr = lax.broadcasted_iota(jnp.int32, (D, D), 0)
        cc = lax.broadcasted_iota(jnp.int32, (D, D), 1)
        diag = jnp.where(rr == cc, G, 0.0)
        drow = jnp.sum(diag, axis=1, keepdims=True)
        dcol = jnp.sum(diag, axis=0, keepdims=True)
        crow = 1.0 / jnp.maximum(jnp.sqrt(drow), 1e-12)
        ccol = 1.0 / jnp.maximum(jnp.sqrt(dcol), 1e-12)
        m_ref[...] = (crow * crow) * G * ccol


def _k3_body(h_ref, h0_ref, m_ref, agg_ref, o_ref):
    o_ref[...] = jnp.maximum(
        ALPHA * h0_ref[...]
        - BETA * jnp.dot(h_ref[...], m_ref[...],
                         preferred_element_type=jnp.float32)
        + agg_ref[0] + agg_ref[1],
        0.0)


def _mm_body(h_ref, wf_ref, bf_ref, o_ref):
    o_ref[...] = (jnp.dot(h_ref[...], wf_ref[...],
                          preferred_element_type=jnp.float32) + bf_ref[...])


def _k1(x, W_init, b_init, d0, d1):
    return pl.pallas_call(
        _k1_body,
        grid=(RB,),
        in_specs=[
            pl.BlockSpec((BM, D), lambda i: (i, 0)),
            pl.BlockSpec((D, D), lambda i: (0, 0)),
            pl.BlockSpec((1, D), lambda i: (0, 0)),
            pl.BlockSpec((RB, BM), lambda i: (0, 0)),
            pl.BlockSpec((RB, BM), lambda i: (0, 0)),
        ],
        out_specs=[
            pl.BlockSpec((BM, D), lambda i: (i, 0)),
            pl.BlockSpec((RB, BM), lambda i: (0, 0)),
        ],
        out_shape=[
            jax.ShapeDtypeStruct((N, D), jnp.float32),
            jax.ShapeDtypeStruct((RB, BM), jnp.float32),
        ],
    )(x, W_init, b_init.reshape(1, D), d0.reshape(RB, BM), d1.reshape(RB, BM))


def _k2(h):
    return pl.pallas_call(
        _k2_body,
        grid=(RB,),
        in_specs=[pl.BlockSpec((BM, D), lambda i: (i, 0))],
        out_specs=pl.BlockSpec((D, D), lambda i: (0, 0)),
        out_shape=jax.ShapeDtypeStruct((D, D), jnp.float32),
        scratch_shapes=[pltpu.VMEM((D, D), jnp.float32)],
    )(h)


def _k3(h, h0, M, agg2):
    return pl.pallas_call(
        _k3_body,
        grid=(RB,),
        in_specs=[
            pl.BlockSpec((BM, D), lambda i: (i, 0)),
            pl.BlockSpec((BM, D), lambda i: (i, 0)),
            pl.BlockSpec((D, D), lambda i: (0, 0)),
            pl.BlockSpec((NC, BM, D), lambda i: (0, i, 0)),
        ],
        out_specs=pl.BlockSpec((BM, D), lambda i: (i, 0)),
        out_shape=jax.ShapeDtypeStruct((N, D), jnp.float32),
    )(h, h0, M, agg2)


def _k4(h, W_final, b_final):
    return pl.pallas_call(
        _mm_body,
        grid=(RB,),
        in_specs=[
            pl.BlockSpec((BM, D), lambda i: (i, 0)),
            pl.BlockSpec((D, D), lambda i: (0, 0)),
            pl.BlockSpec((1, D), lambda i: (0, 0)),
        ],
        out_specs=pl.BlockSpec((BM, D), lambda i: (i, 0)),
        out_shape=jax.ShapeDtypeStruct((N, D), jnp.float32),
    )(h, W_final, b_final.reshape(1, D))


def kernel(x, edge_index, diffusion_distance, W_init, b_init,
           W_final, b_final):
    src_tiles = edge_index[0].reshape(NT, NCH, CH)
    dst_tiles = edge_index[1].reshape(NT, NCH, CH)
    dd_tiles = diffusion_distance.reshape(NT, NCH, CH)

    d0, d1 = _deg_partials(dst_tiles)
    h, w2 = _k1(x, W_init, b_init, d0, d1)
    w = w2.reshape(N)
    h0 = h
    for _ in range(NUM_LAYERS):
        M = _k2(h)
        agg2 = _edge_partials(h, w, src_tiles, dst_tiles, dd_tiles)
        h = _k3(h, h0, M, agg2)
    return _k4(h, W_final, b_final)


# speedup vs baseline: 1.2551x; 1.0216x over previous
import dataclasses

import jax
import jax.numpy as jnp
from jax import lax
from jax.experimental import pallas as pl
from jax.experimental.pallas import tpu as pltpu
from jax.experimental.pallas import tpu_sc as plsc

N = 10000
D = 128
E = 320000
NUM_LAYERS = 2
ALPHA = 0.1
BETA = 0.1
ETA = 0.5
EPS = 1e-5

NC = 2
NS = 16
NT = NC * NS
PER_TILE = E // NT
CH = 80
NCH = PER_TILE // CH
RPS = 624
TAIL = N - NS * RPS
RB = 5
BM = N // RB

_mesh = plsc.VectorSubcoreMesh(core_axis_name="c", subcore_axis_name="s")

_sc_params = pltpu.CompilerParams()
if "needs_layout_passes" in pltpu.CompilerParams.__dataclass_fields__:
    _sc_params = dataclasses.replace(_sc_params, needs_layout_passes=False)


def _deg_body(dst_hbm, out0_hbm, out1_hbm, deg_sh, dst_v, ones_v, zero_v):
    ci = lax.axis_index("c")
    si = lax.axis_index("s")
    tid = ci * NS + si

    @pl.loop(0, CH // 16)
    def _(i):
        ones_v[pl.ds(i * 16, 16)] = jnp.ones((16,), jnp.float32)

    @pl.when(si == 0)
    def _():
        @pl.loop(0, N // 16)
        def _(i):
            zero_v[pl.ds(i * 16, 16)] = jnp.zeros((16,), jnp.float32)
        pltpu.sync_copy(zero_v, deg_sh)

    pltpu.sync_copy(dst_hbm.at[tid], dst_v)
    plsc.subcore_barrier()

    @pl.loop(0, NCH)
    def _(c):
        pltpu.sync_copy(ones_v, deg_sh.at[dst_v.at[c]], add=True)

    plsc.subcore_barrier()

    @pl.when(jnp.logical_and(si == 0, ci == 0))
    def _():
        pltpu.sync_copy(deg_sh, out0_hbm)

    @pl.when(jnp.logical_and(si == 0, ci == 1))
    def _():
        pltpu.sync_copy(deg_sh, out1_hbm)


def _edge_body(h_hbm, w_hbm, src_hbm, dst_hbm, dd_hbm, out_hbm,
               agg_sh, w_v, sidx_v, didx_v, ddc_v, hs_v, hd_v, msg_v,
               sem1, sem2):
    ci = lax.axis_index("c")
    si = lax.axis_index("s")
    tid = ci * NS + si

    pltpu.sync_copy(w_hbm, w_v)

    @pl.loop(0, CH)
    def _(r):
        for cc in range(D // 16):
            msg_v[r, pl.ds(cc * 16, 16)] = jnp.zeros((16,), jnp.float32)

    for t in range(RPS // CH):
        pltpu.sync_copy(msg_v, agg_sh.at[pl.ds(si * RPS + t * CH, CH)])
    rem = RPS - (RPS // CH) * CH
    pltpu.sync_copy(msg_v.at[pl.ds(0, rem)],
                    agg_sh.at[pl.ds(si * RPS + RPS - rem, rem)])

    @pl.when(si == NS - 1)
    def _():
        pltpu.sync_copy(msg_v.at[pl.ds(0, TAIL)],
                        agg_sh.at[pl.ds(NS * RPS, TAIL)])

    plsc.subcore_barrier()

    iota16 = lax.iota(jnp.int32, 16)

    @pl.loop(0, NCH)
    def _chunk(c):
        pltpu.sync_copy(src_hbm.at[tid, c], sidx_v)
        pltpu.sync_copy(dst_hbm.at[tid, c], didx_v)
        pltpu.sync_copy(dd_hbm.at[tid, c], ddc_v)
        g1 = pltpu.async_copy(h_hbm.at[sidx_v], hs_v, sem1)
        g2 = pltpu.async_copy(h_hbm.at[didx_v], hd_v, sem2)
        g1.wait()
        g2.wait()
        for g in range(CH // 16):
            rb = g * 16
            src16 = sidx_v[pl.ds(rb, 16)]
            dst16 = didx_v[pl.ds(rb, 16)]
            dd16 = ddc_v[pl.ds(rb, 16)]
            ws = plsc.load_gather(w_v, [src16])
            wd = plsc.load_gather(w_v, [dst16])
            rows = rb + iota16

            def _acc(kk, acc):
                for j in range(16):
                    col = lax.broadcast(kk * 16 + j, (16,))
                    hs_k = plsc.load_gather(hs_v, [rows, col])
                    hd_k = plsc.load_gather(hd_v, [rows, col])
                    t = wd * hd_k - ws * hs_k
                    acc = acc + t * t
                return acc

            s = lax.fori_loop(0, D // 16, _acc, jnp.zeros((16,), jnp.float32))
            scl = jnp.maximum(s, 1e-20)
            ib = lax.bitcast_convert_type(scl, jnp.int32)
            ib = jnp.int32(0x5F3759DF) - lax.shift_right_arithmetic(ib, 1)
            y = lax.bitcast_convert_type(ib, jnp.float32)
            for _ in range(3):
                y = y * (1.5 - 0.5 * scl * y * y)
            denom = scl * y + EPS
            q = (ETA * dd16) / denom
            a16 = ws * wd * (1.0 - q)
            b16 = (ws * ws) * q

            @pl.loop(0, D // 16)
            def _msg(kk):
                for j in range(16):
                    col = lax.broadcast(kk * 16 + j, (16,))
                    hs_k = plsc.load_gather(hs_v, [rows, col])
                    hd_k = plsc.load_gather(hd_v, [rows, col])
                    plsc.store_scatter(msg_v, [rows, col],
                                       a16 * hs_k + b16 * hd_k)

        pltpu.sync_copy(msg_v, agg_sh.at[didx_v], add=True)

    plsc.subcore_barrier()
    pltpu.sync_copy(agg_sh.at[pl.ds(si * RPS, RPS)],
                    out_hbm.at[ci, pl.ds(si * RPS, RPS)])

    @pl.when(si == NS - 1)
    def _():
        pltpu.sync_copy(agg_sh.at[pl.ds(NS * RPS, TAIL)],
                        out_hbm.at[ci, pl.ds(NS * RPS, TAIL)])


def _deg_partials(dst_tiles):
    return pl.kernel(
        _deg_body,
        out_type=[jax.ShapeDtypeStruct((N,), jnp.float32),
                  jax.ShapeDtypeStruct((N,), jnp.float32)],
        mesh=_mesh,
        compiler_params=_sc_params,
        scratch_types=[
            pltpu.VMEM_SHARED((N,), jnp.float32),
            pltpu.VMEM((NCH, CH), jnp.int32),
            pltpu.VMEM((CH,), jnp.float32),
            pltpu.VMEM((N,), jnp.float32),
        ],
    )(dst_tiles)


def _edge_partials(h, w, src_tiles, dst_tiles, dd_tiles):
    return pl.kernel(
        _edge_body,
        out_type=jax.ShapeDtypeStruct((NC, N, D), jnp.float32),
        mesh=_mesh,
        compiler_params=_sc_params,
        scratch_types=[
            pltpu.VMEM_SHARED((N, D), jnp.float32),
            pltpu.VMEM((N,), jnp.float32),
            pltpu.VMEM((CH,), jnp.int32),
            pltpu.VMEM((CH,), jnp.int32),
            pltpu.VMEM((CH,), jnp.float32),
            pltpu.VMEM((CH, D), jnp.float32),
            pltpu.VMEM((CH, D), jnp.float32),
            pltpu.VMEM((CH, D), jnp.float32),
            pltpu.SemaphoreType.DMA,
            pltpu.SemaphoreType.DMA,
        ],
    )(h, w, src_tiles, dst_tiles, dd_tiles)


def _k1_body(x_ref, wi_ref, bi_ref, d0_ref, d1_ref, h_ref, w_ref):
    h_ref[...] = (jnp.dot(x_ref[...], wi_ref[...],
                          preferred_element_type=jnp.float32) + bi_ref[...])
    deg = d0_ref[...] + d1_ref[...]
    w_ref[...] = jnp.where(deg > 0.0, lax.rsqrt(jnp.maximum(deg, 1.0)), 0.0)


def _k2_body(h_ref, m_ref, acc_ref):
    i = pl.program_id(0)

    @pl.when(i == 0)
    def _():
        acc_ref[...] = jnp.zeros_like(acc_ref)

    hb = h_ref[...]
    acc_ref[...] += lax.dot_general(hb, hb, (((0,), (0,)), ((), ())),
                                    preferred_element_type=jnp.float32)

    @pl.when(i == RB - 1)
    def _():
        G = acc_ref[...]
        rr = lax.broadcasted_iota(jnp.int32, (D, D), 0)
        cc = lax.broadcasted_iota(jnp.int32, (D, D), 1)
        diag = jnp.where(rr == cc, G, 0.0)
        drow = jnp.sum(diag, axis=1, keepdims=True)
        dcol = jnp.sum(diag, axis=0, keepdims=True)
        crow = 1.0 / jnp.maximum(jnp.sqrt(drow), 1e-12)
        ccol = 1.0 / jnp.maximum(jnp.sqrt(dcol), 1e-12)
        m_ref[...] = (crow * crow) * G * ccol


def _k3_body(h_ref, h0_ref, m_ref, agg_ref, o_ref):
    o_ref[...] = jnp.maximum(
        ALPHA * h0_ref[...]
        - BETA * jnp.dot(h_ref[...], m_ref[...],
                         preferred_element_type=jnp.float32)
        + agg_ref[0] + agg_ref[1],
        0.0)


def _mm_body(h_ref, wf_ref, bf_ref, o_ref):
    o_ref[...] = (jnp.dot(h_ref[...], wf_ref[...],
                          preferred_element_type=jnp.float32) + bf_ref[...])


def _k1(x, W_init, b_init, d0, d1):
    return pl.pallas_call(
        _k1_body,
        grid=(RB,),
        in_specs=[
            pl.BlockSpec((BM, D), lambda i: (i, 0)),
            pl.BlockSpec((D, D), lambda i: (0, 0)),
            pl.BlockSpec((1, D), lambda i: (0, 0)),
            pl.BlockSpec((RB, BM), lambda i: (0, 0)),
            pl.BlockSpec((RB, BM), lambda i: (0, 0)),
        ],
        out_specs=[
            pl.BlockSpec((BM, D), lambda i: (i, 0)),
            pl.BlockSpec((RB, BM), lambda i: (0, 0)),
        ],
        out_shape=[
            jax.ShapeDtypeStruct((N, D), jnp.float32),
            jax.ShapeDtypeStruct((RB, BM), jnp.float32),
        ],
    )(x, W_init, b_init.reshape(1, D), d0.reshape(RB, BM), d1.reshape(RB, BM))


def _k2(h):
    return pl.pallas_call(
        _k2_body,
        grid=(RB,),
        in_specs=[pl.BlockSpec((BM, D), lambda i: (i, 0))],
        out_specs=pl.BlockSpec((D, D), lambda i: (0, 0)),
        out_shape=jax.ShapeDtypeStruct((D, D), jnp.float32),
        scratch_shapes=[pltpu.VMEM((D, D), jnp.float32)],
    )(h)


def _k3(h, h0, M, agg2):
    return pl.pallas_call(
        _k3_body,
        grid=(RB,),
        in_specs=[
            pl.BlockSpec((BM, D), lambda i: (i, 0)),
            pl.BlockSpec((BM, D), lambda i: (i, 0)),
            pl.BlockSpec((D, D), lambda i: (0, 0)),
            pl.BlockSpec((NC, BM, D), lambda i: (0, i, 0)),
        ],
        out_specs=pl.BlockSpec((BM, D), lambda i: (i, 0)),
        out_shape=jax.ShapeDtypeStruct((N, D), jnp.float32),
    )(h, h0, M, agg2)


def _k4(h, W_final, b_final):
    return pl.pallas_call(
        _mm_body,
        grid=(RB,),
        in_specs=[
            pl.BlockSpec((BM, D), lambda i: (i, 0)),
            pl.BlockSpec((D, D), lambda i: (0, 0)),
            pl.BlockSpec((1, D), lambda i: (0, 0)),
        ],
        out_specs=pl.BlockSpec((BM, D), lambda i: (i, 0)),
        out_shape=jax.ShapeDtypeStruct((N, D), jnp.float32),
    )(h, W_final, b_final.reshape(1, D))


def kernel(x, edge_index, diffusion_distance, W_init, b_init,
           W_final, b_final):
    src_tiles = edge_index[0].reshape(NT, NCH, CH)
    dst_tiles = edge_index[1].reshape(NT, NCH, CH)
    dd_tiles = diffusion_distance.reshape(NT, NCH, CH)

    d0, d1 = _deg_partials(dst_tiles)
    h, w2 = _k1(x, W_init, b_init, d0, d1)
    w = w2.reshape(N)
    h0 = h
    for _ in range(NUM_LAYERS):
        M = _k2(h)
        agg2 = _edge_partials(h, w, src_tiles, dst_tiles, dd_tiles)
        h = _k3(h, h0, M, agg2)
    return _k4(h, W_final, b_final)


# speedup vs baseline: 4.8096x; 3.8320x over previous
import dataclasses

import jax
import jax.numpy as jnp
from jax import lax
from jax.experimental import pallas as pl
from jax.experimental.pallas import tpu as pltpu
from jax.experimental.pallas import tpu_sc as plsc

N = 10000
D = 128
E = 320000
NUM_LAYERS = 2
ALPHA = 0.1
BETA = 0.1
ETA = 0.5
EPS = 1e-5

NC = 2
NS = 16
NT = NC * NS
PER_TILE = E // NT
CH = 80
NCH = PER_TILE // CH
RPS = 624
TAIL = N - NS * RPS
RB = 5
BM = N // RB

_mesh = plsc.VectorSubcoreMesh(core_axis_name="c", subcore_axis_name="s")

_sc_params = pltpu.CompilerParams()
if "needs_layout_passes" in pltpu.CompilerParams.__dataclass_fields__:
    _sc_params = dataclasses.replace(_sc_params, needs_layout_passes=False)


def _deg_body(dst_hbm, out0_hbm, out1_hbm, deg_sh, dst_v, ones_v, zero_v):
    ci = lax.axis_index("c")
    si = lax.axis_index("s")
    tid = ci * NS + si

    @pl.loop(0, CH // 16)
    def _(i):
        ones_v[pl.ds(i * 16, 16)] = jnp.ones((16,), jnp.float32)

    @pl.when(si == 0)
    def _():
        @pl.loop(0, N // 16)
        def _(i):
            zero_v[pl.ds(i * 16, 16)] = jnp.zeros((16,), jnp.float32)
        pltpu.sync_copy(zero_v, deg_sh)

    pltpu.sync_copy(dst_hbm.at[tid], dst_v)
    plsc.subcore_barrier()

    @pl.loop(0, NCH)
    def _(c):
        pltpu.sync_copy(ones_v, deg_sh.at[dst_v.at[c]], add=True)

    plsc.subcore_barrier()

    @pl.when(jnp.logical_and(si == 0, ci == 0))
    def _():
        pltpu.sync_copy(deg_sh, out0_hbm)

    @pl.when(jnp.logical_and(si == 0, ci == 1))
    def _():
        pltpu.sync_copy(deg_sh, out1_hbm)


def _edge_body(h_hbm, w_hbm, src_hbm, dst_hbm, dd_hbm, out_hbm,
               agg_sh, w_v, sidx_v, didx_v, ddc_v, ws_v, wd_v,
               hs_v, hd_v, msg_v, sem1, sem2):
    ci = lax.axis_index("c")
    si = lax.axis_index("s")
    tid = ci * NS + si

    pltpu.sync_copy(w_hbm, w_v)

    @pl.loop(0, CH)
    def _(r):
        for cc in range(D // 16):
            msg_v[r, pl.ds(cc * 16, 16)] = jnp.zeros((16,), jnp.float32)

    for t in range(RPS // CH):
        pltpu.sync_copy(msg_v, agg_sh.at[pl.ds(si * RPS + t * CH, CH)])
    rem = RPS - (RPS // CH) * CH
    pltpu.sync_copy(msg_v.at[pl.ds(0, rem)],
                    agg_sh.at[pl.ds(si * RPS + RPS - rem, rem)])

    @pl.when(si == NS - 1)
    def _():
        pltpu.sync_copy(msg_v.at[pl.ds(0, TAIL)],
                        agg_sh.at[pl.ds(NS * RPS, TAIL)])

    plsc.subcore_barrier()

    @pl.loop(0, NCH)
    def _chunk(c):
        pltpu.sync_copy(src_hbm.at[tid, c], sidx_v)
        pltpu.sync_copy(dst_hbm.at[tid, c], didx_v)
        pltpu.sync_copy(dd_hbm.at[tid, c], ddc_v)
        g1 = pltpu.async_copy(h_hbm.at[sidx_v], hs_v, sem1)
        g2 = pltpu.async_copy(h_hbm.at[didx_v], hd_v, sem2)
        for g in range(CH // 16):
            rb = g * 16
            src16 = sidx_v[pl.ds(rb, 16)]
            dst16 = didx_v[pl.ds(rb, 16)]
            ws_v[pl.ds(rb, 16)] = plsc.load_gather(w_v, [src16])
            wd_v[pl.ds(rb, 16)] = plsc.load_gather(w_v, [dst16])
        g1.wait()
        g2.wait()

        @pl.loop(0, CH)
        def _edge(e):
            ev = lax.broadcast(e, (16,))
            ws = plsc.load_gather(ws_v, [ev])
            wd = plsc.load_gather(wd_v, [ev])
            dd = plsc.load_gather(ddc_v, [ev])
            hs_c = [hs_v[e, pl.ds(k * 16, 16)] for k in range(D // 16)]
            hd_c = [hd_v[e, pl.ds(k * 16, 16)] for k in range(D // 16)]
            acc = jnp.zeros((16,), jnp.float32)
            for k in range(D // 16):
                t = wd * hd_c[k] - ws * hs_c[k]
                acc = acc + t * t
            s = lax.broadcast(jnp.sum(acc), (16,))
            scl = jnp.maximum(s, 1e-20)
            ib = lax.bitcast_convert_type(scl, jnp.int32)
            ib = jnp.int32(0x5F3759DF) - lax.shift_right_arithmetic(ib, 1)
            y = lax.bitcast_convert_type(ib, jnp.float32)
            for _ in range(3):
                y = y * (1.5 - 0.5 * scl * y * y)
            denom = scl * y + EPS
            q = (ETA * dd) / denom
            a = ws * wd * (1.0 - q)
            b = (ws * ws) * q
            for k in range(D // 16):
                msg_v[e, pl.ds(k * 16, 16)] = a * hs_c[k] + b * hd_c[k]

        pltpu.sync_copy(msg_v, agg_sh.at[didx_v], add=True)

    plsc.subcore_barrier()
    pltpu.sync_copy(agg_sh.at[pl.ds(si * RPS, RPS)],
                    out_hbm.at[ci, pl.ds(si * RPS, RPS)])

    @pl.when(si == NS - 1)
    def _():
        pltpu.sync_copy(agg_sh.at[pl.ds(NS * RPS, TAIL)],
                        out_hbm.at[ci, pl.ds(NS * RPS, TAIL)])


def _deg_partials(dst_tiles):
    return pl.kernel(
        _deg_body,
        out_type=[jax.ShapeDtypeStruct((N,), jnp.float32),
                  jax.ShapeDtypeStruct((N,), jnp.float32)],
        mesh=_mesh,
        compiler_params=_sc_params,
        scratch_types=[
            pltpu.VMEM_SHARED((N,), jnp.float32),
            pltpu.VMEM((NCH, CH), jnp.int32),
            pltpu.VMEM((CH,), jnp.float32),
            pltpu.VMEM((N,), jnp.float32),
        ],
    )(dst_tiles)


def _edge_partials(h, w, src_tiles, dst_tiles, dd_tiles):
    return pl.kernel(
        _edge_body,
        out_type=jax.ShapeDtypeStruct((NC, N, D), jnp.float32),
        mesh=_mesh,
        compiler_params=_sc_params,
        scratch_types=[
            pltpu.VMEM_SHARED((N, D), jnp.float32),
            pltpu.VMEM((N,), jnp.float32),
            pltpu.VMEM((CH,), jnp.int32),
            pltpu.VMEM((CH,), jnp.int32),
            pltpu.VMEM((CH,), jnp.float32),
            pltpu.VMEM((CH,), jnp.float32),
            pltpu.VMEM((CH,), jnp.float32),
            pltpu.VMEM((CH, D), jnp.float32),
            pltpu.VMEM((CH, D), jnp.float32),
            pltpu.VMEM((CH, D), jnp.float32),
            pltpu.SemaphoreType.DMA,
            pltpu.SemaphoreType.DMA,
        ],
    )(h, w, src_tiles, dst_tiles, dd_tiles)


def _k1_body(x_ref, wi_ref, bi_ref, d0_ref, d1_ref, h_ref, w_ref):
    h_ref[...] = (jnp.dot(x_ref[...], wi_ref[...],
                          preferred_element_type=jnp.float32) + bi_ref[...])
    deg = d0_ref[...] + d1_ref[...]
    w_ref[...] = jnp.where(deg > 0.0, lax.rsqrt(jnp.maximum(deg, 1.0)), 0.0)


def _k2_body(h_ref, m_ref, acc_ref):
    i = pl.program_id(0)

    @pl.when(i == 0)
    def _():
        acc_ref[...] = jnp.zeros_like(acc_ref)

    hb = h_ref[...]
    acc_ref[...] += lax.dot_general(hb, hb, (((0,), (0,)), ((), ())),
                                    preferred_element_type=jnp.float32)

    @pl.when(i == RB - 1)
    def _():
        G = acc_ref[...]
        rr = lax.broadcasted_iota(jnp.int32, (D, D), 0)
        cc = lax.broadcasted_iota(jnp.int32, (D, D), 1)
        diag = jnp.where(rr == cc, G, 0.0)
        drow = jnp.sum(diag, axis=1, keepdims=True)
        dcol = jnp.sum(diag, axis=0, keepdims=True)
        crow = 1.0 / jnp.maximum(jnp.sqrt(drow), 1e-12)
        ccol = 1.0 / jnp.maximum(jnp.sqrt(dcol), 1e-12)
        m_ref[...] = (crow * crow) * G * ccol


def _k3_body(h_ref, h0_ref, m_ref, agg_ref, o_ref):
    o_ref[...] = jnp.maximum(
        ALPHA * h0_ref[...]
        - BETA * jnp.dot(h_ref[...], m_ref[...],
                         preferred_element_type=jnp.float32)
        + agg_ref[0] + agg_ref[1],
        0.0)


def _mm_body(h_ref, wf_ref, bf_ref, o_ref):
    o_ref[...] = (jnp.dot(h_ref[...], wf_ref[...],
                          preferred_element_type=jnp.float32) + bf_ref[...])


def _k1(x, W_init, b_init, d0, d1):
    return pl.pallas_call(
        _k1_body,
        grid=(RB,),
        in_specs=[
            pl.BlockSpec((BM, D), lambda i: (i, 0)),
            pl.BlockSpec((D, D), lambda i: (0, 0)),
            pl.BlockSpec((1, D), lambda i: (0, 0)),
            pl.BlockSpec((RB, BM), lambda i: (0, 0)),
            pl.BlockSpec((RB, BM), lambda i: (0, 0)),
        ],
        out_specs=[
            pl.BlockSpec((BM, D), lambda i: (i, 0)),
            pl.BlockSpec((RB, BM), lambda i: (0, 0)),
        ],
        out_shape=[
            jax.ShapeDtypeStruct((N, D), jnp.float32),
            jax.ShapeDtypeStruct((RB, BM), jnp.float32),
        ],
    )(x, W_init, b_init.reshape(1, D), d0.reshape(RB, BM), d1.reshape(RB, BM))


def _k2(h):
    return pl.pallas_call(
        _k2_body,
        grid=(RB,),
        in_specs=[pl.BlockSpec((BM, D), lambda i: (i, 0))],
        out_specs=pl.BlockSpec((D, D), lambda i: (0, 0)),
        out_shape=jax.ShapeDtypeStruct((D, D), jnp.float32),
        scratch_shapes=[pltpu.VMEM((D, D), jnp.float32)],
    )(h)


def _k3(h, h0, M, agg2):
    return pl.pallas_call(
        _k3_body,
        grid=(RB,),
        in_specs=[
            pl.BlockSpec((BM, D), lambda i: (i, 0)),
            pl.BlockSpec((BM, D), lambda i: (i, 0)),
            pl.BlockSpec((D, D), lambda i: (0, 0)),
            pl.BlockSpec((NC, BM, D), lambda i: (0, i, 0)),
        ],
        out_specs=pl.BlockSpec((BM, D), lambda i: (i, 0)),
        out_shape=jax.ShapeDtypeStruct((N, D), jnp.float32),
    )(h, h0, M, agg2)


def _k4(h, W_final, b_final):
    return pl.pallas_call(
        _mm_body,
        grid=(RB,),
        in_specs=[
            pl.BlockSpec((BM, D), lambda i: (i, 0)),
            pl.BlockSpec((D, D), lambda i: (0, 0)),
            pl.BlockSpec((1, D), lambda i: (0, 0)),
        ],
        out_specs=pl.BlockSpec((BM, D), lambda i: (i, 0)),
        out_shape=jax.ShapeDtypeStruct((N, D), jnp.float32),
    )(h, W_final, b_final.reshape(1, D))


def kernel(x, edge_index, diffusion_distance, W_init, b_init,
           W_final, b_final):
    src_tiles = edge_index[0].reshape(NT, NCH, CH)
    dst_tiles = edge_index[1].reshape(NT, NCH, CH)
    dd_tiles = diffusion_distance.reshape(NT, NCH, CH)

    d0, d1 = _deg_partials(dst_tiles)
    h, w2 = _k1(x, W_init, b_init, d0, d1)
    w = w2.reshape(N)
    h0 = h
    for _ in range(NUM_LAYERS):
        M = _k2(h)
        agg2 = _edge_partials(h, w, src_tiles, dst_tiles, dd_tiles)
        h = _k3(h, h0, M, agg2)
    return _k4(h, W_final, b_final)


# speedup vs baseline: 5.0633x; 1.0527x over previous
import dataclasses

import jax
import jax.numpy as jnp
from jax import lax
from jax.experimental import pallas as pl
from jax.experimental.pallas import tpu as pltpu
from jax.experimental.pallas import tpu_sc as plsc

N = 10000
D = 128
E = 320000
NUM_LAYERS = 2
ALPHA = 0.1
BETA = 0.1
ETA = 0.5
EPS = 1e-5

NC = 2
NS = 16
NT = NC * NS
PER_TILE = E // NT
CH = 80
NCH = PER_TILE // CH
RPS = 624
TAIL = N - NS * RPS
RB = 5
BM = N // RB

_mesh = plsc.VectorSubcoreMesh(core_axis_name="c", subcore_axis_name="s")

_sc_params = pltpu.CompilerParams()
if "needs_layout_passes" in pltpu.CompilerParams.__dataclass_fields__:
    _sc_params = dataclasses.replace(_sc_params, needs_layout_passes=False)


def _deg_body(dst_hbm, out0_hbm, out1_hbm, deg_sh, dst_v, ones_v, zero_v):
    ci = lax.axis_index("c")
    si = lax.axis_index("s")
    tid = ci * NS + si

    @pl.loop(0, CH // 16)
    def _(i):
        ones_v[pl.ds(i * 16, 16)] = jnp.ones((16,), jnp.float32)

    @pl.when(si == 0)
    def _():
        @pl.loop(0, N // 16)
        def _(i):
            zero_v[pl.ds(i * 16, 16)] = jnp.zeros((16,), jnp.float32)
        pltpu.sync_copy(zero_v, deg_sh)

    pltpu.sync_copy(dst_hbm.at[tid], dst_v)
    plsc.subcore_barrier()

    @pl.loop(0, NCH)
    def _(c):
        pltpu.sync_copy(ones_v, deg_sh.at[dst_v.at[c]], add=True)

    plsc.subcore_barrier()

    @pl.when(jnp.logical_and(si == 0, ci == 0))
    def _():
        pltpu.sync_copy(deg_sh, out0_hbm)

    @pl.when(jnp.logical_and(si == 0, ci == 1))
    def _():
        pltpu.sync_copy(deg_sh, out1_hbm)


def _edge_body(h_hbm, w_hbm, src_hbm, dst_hbm, dd_hbm, out_hbm,
               agg_sh, w_v, sidx_v, didx_v, didx2_v, ddc_v, ws_v, wd_v,
               hs_v, hd_v, msg_v, sem1, sem2, sem3, sem4):
    ci = lax.axis_index("c")
    si = lax.axis_index("s")
    tid = ci * NS + si

    pltpu.sync_copy(w_hbm, w_v)

    @pl.loop(0, CH)
    def _(r):
        for cc in range(D // 16):
            msg_v[r, pl.ds(cc * 16, 16)] = jnp.zeros((16,), jnp.float32)

    for t in range(RPS // CH):
        pltpu.sync_copy(msg_v, agg_sh.at[pl.ds(si * RPS + t * CH, CH)])
    rem = RPS - (RPS // CH) * CH
    pltpu.sync_copy(msg_v.at[pl.ds(0, rem)],
                    agg_sh.at[pl.ds(si * RPS + RPS - rem, rem)])

    @pl.when(si == NS - 1)
    def _():
        pltpu.sync_copy(msg_v.at[pl.ds(0, TAIL)],
                        agg_sh.at[pl.ds(NS * RPS, TAIL)])

    plsc.subcore_barrier()

    def _do_chunk(c, hsbuf, dibuf, scsem):
        pltpu.sync_copy(src_hbm.at[tid, c], sidx_v)
        pltpu.sync_copy(dst_hbm.at[tid, c], dibuf)
        pltpu.sync_copy(dd_hbm.at[tid, c], ddc_v)
        g1 = pltpu.async_copy(h_hbm.at[sidx_v], hsbuf, sem1)
        g2 = pltpu.async_copy(h_hbm.at[dibuf], hd_v, sem2)
        for g in range(CH // 16):
            rb = g * 16
            src16 = sidx_v[pl.ds(rb, 16)]
            dst16 = dibuf[pl.ds(rb, 16)]
            ws_v[pl.ds(rb, 16)] = plsc.load_gather(w_v, [src16])
            wd_v[pl.ds(rb, 16)] = plsc.load_gather(w_v, [dst16])
        g1.wait()
        g2.wait()

        @pl.loop(0, CH)
        def _edge(e):
            ev = lax.broadcast(e, (16,))
            ws = plsc.load_gather(ws_v, [ev])
            wd = plsc.load_gather(wd_v, [ev])
            dd = plsc.load_gather(ddc_v, [ev])
            hs_c = [hsbuf[e, pl.ds(k * 16, 16)] for k in range(D // 16)]
            hd_c = [hd_v[e, pl.ds(k * 16, 16)] for k in range(D // 16)]
            acc = jnp.zeros((16,), jnp.float32)
            for k in range(D // 16):
                t = wd * hd_c[k] - ws * hs_c[k]
                acc = acc + t * t
            s = lax.broadcast(jnp.sum(acc), (16,))
            scl = jnp.maximum(s, 1e-20)
            ib = lax.bitcast_convert_type(scl, jnp.int32)
            ib = jnp.int32(0x5F3759DF) - lax.shift_right_arithmetic(ib, 1)
            y = lax.bitcast_convert_type(ib, jnp.float32)
            for _ in range(3):
                y = y * (1.5 - 0.5 * scl * y * y)
            denom = scl * y + EPS
            q = (ETA * dd) / denom
            a = ws * wd * (1.0 - q)
            b = (ws * ws) * q
            for k in range(D // 16):
                hsbuf[e, pl.ds(k * 16, 16)] = a * hs_c[k] + b * hd_c[k]

        return pltpu.async_copy(hsbuf, agg_sh.at[dibuf], scsem, add=True)

    @pl.loop(0, NCH // 2)
    def _pair(p):
        s_even = _do_chunk(2 * p, hs_v, didx_v, sem3)
        s_odd = _do_chunk(2 * p + 1, msg_v, didx2_v, sem4)
        s_even.wait()

        @pl.when(p == NCH // 2 - 1)
        def _():
            _do_chunk(2 * p + 2, hs_v, didx_v, sem3).wait()

        s_odd.wait()

    plsc.subcore_barrier()
    pltpu.sync_copy(agg_sh.at[pl.ds(si * RPS, RPS)],
                    out_hbm.at[ci, pl.ds(si * RPS, RPS)])

    @pl.when(si == NS - 1)
    def _():
        pltpu.sync_copy(agg_sh.at[pl.ds(NS * RPS, TAIL)],
                        out_hbm.at[ci, pl.ds(NS * RPS, TAIL)])


def _deg_partials(dst_tiles):
    return pl.kernel(
        _deg_body,
        out_type=[jax.ShapeDtypeStruct((N,), jnp.float32),
                  jax.ShapeDtypeStruct((N,), jnp.float32)],
        mesh=_mesh,
        compiler_params=_sc_params,
        scratch_types=[
            pltpu.VMEM_SHARED((N,), jnp.float32),
            pltpu.VMEM((NCH, CH), jnp.int32),
            pltpu.VMEM((CH,), jnp.float32),
            pltpu.VMEM((N,), jnp.float32),
        ],
    )(dst_tiles)


def _edge_partials(h, w, src_tiles, dst_tiles, dd_tiles):
    return pl.kernel(
        _edge_body,
        out_type=jax.ShapeDtypeStruct((NC, N, D), jnp.float32),
        mesh=_mesh,
        compiler_params=_sc_params,
        scratch_types=[
            pltpu.VMEM_SHARED((N, D), jnp.float32),
            pltpu.VMEM((N,), jnp.float32),
            pltpu.VMEM((CH,), jnp.int32),
            pltpu.VMEM((CH,), jnp.int32),
            pltpu.VMEM((CH,), jnp.int32),
            pltpu.VMEM((CH,), jnp.float32),
            pltpu.VMEM((CH,), jnp.float32),
            pltpu.VMEM((CH,), jnp.float32),
            pltpu.VMEM((CH, D), jnp.float32),
            pltpu.VMEM((CH, D), jnp.float32),
            pltpu.VMEM((CH, D), jnp.float32),
            pltpu.SemaphoreType.DMA,
            pltpu.SemaphoreType.DMA,
            pltpu.SemaphoreType.DMA,
            pltpu.SemaphoreType.DMA,
        ],
    )(h, w, src_tiles, dst_tiles, dd_tiles)


def _k1_body(x_ref, wi_ref, bi_ref, d0_ref, d1_ref, h_ref, w_ref):
    h_ref[...] = (jnp.dot(x_ref[...], wi_ref[...],
                          preferred_element_type=jnp.float32) + bi_ref[...])
    deg = d0_ref[...] + d1_ref[...]
    w_ref[...] = jnp.where(deg > 0.0, lax.rsqrt(jnp.maximum(deg, 1.0)), 0.0)


def _k2_body(h_ref, m_ref, acc_ref):
    i = pl.program_id(0)

    @pl.when(i == 0)
    def _():
        acc_ref[...] = jnp.zeros_like(acc_ref)

    hb = h_ref[...]
    acc_ref[...] += lax.dot_general(hb, hb, (((0,), (0,)), ((), ())),
                                    preferred_element_type=jnp.float32)

    @pl.when(i == RB - 1)
    def _():
        G = acc_ref[...]
        rr = lax.broadcasted_iota(jnp.int32, (D, D), 0)
        cc = lax.broadcasted_iota(jnp.int32, (D, D), 1)
        diag = jnp.where(rr == cc, G, 0.0)
        drow = jnp.sum(diag, axis=1, keepdims=True)
        dcol = jnp.sum(diag, axis=0, keepdims=True)
        crow = 1.0 / jnp.maximum(jnp.sqrt(drow), 1e-12)
        ccol = 1.0 / jnp.maximum(jnp.sqrt(dcol), 1e-12)
        m_ref[...] = (crow * crow) * G * ccol


def _k3_body(h_ref, h0_ref, m_ref, agg_ref, o_ref):
    o_ref[...] = jnp.maximum(
        ALPHA * h0_ref[...]
        - BETA * jnp.dot(h_ref[...], m_ref[...],
                         preferred_element_type=jnp.float32)
        + agg_ref[0] + agg_ref[1],
        0.0)


def _mm_body(h_ref, wf_ref, bf_ref, o_ref):
    o_ref[...] = (jnp.dot(h_ref[...], wf_ref[...],
                          preferred_element_type=jnp.float32) + bf_ref[...])


def _k1(x, W_init, b_init, d0, d1):
    return pl.pallas_call(
        _k1_body,
        grid=(RB,),
        in_specs=[
            pl.BlockSpec((BM, D), lambda i: (i, 0)),
            pl.BlockSpec((D, D), lambda i: (0, 0)),
            pl.BlockSpec((1, D), lambda i: (0, 0)),
            pl.BlockSpec((RB, BM), lambda i: (0, 0)),
            pl.BlockSpec((RB, BM), lambda i: (0, 0)),
        ],
        out_specs=[
            pl.BlockSpec((BM, D), lambda i: (i, 0)),
            pl.BlockSpec((RB, BM), lambda i: (0, 0)),
        ],
        out_shape=[
            jax.ShapeDtypeStruct((N, D), jnp.float32),
            jax.ShapeDtypeStruct((RB, BM), jnp.float32),
        ],
    )(x, W_init, b_init.reshape(1, D), d0.reshape(RB, BM), d1.reshape(RB, BM))


def _k2(h):
    return pl.pallas_call(
        _k2_body,
        grid=(RB,),
        in_specs=[pl.BlockSpec((BM, D), lambda i: (i, 0))],
        out_specs=pl.BlockSpec((D, D), lambda i: (0, 0)),
        out_shape=jax.ShapeDtypeStruct((D, D), jnp.float32),
        scratch_shapes=[pltpu.VMEM((D, D), jnp.float32)],
    )(h)


def _k3(h, h0, M, agg2):
    return pl.pallas_call(
        _k3_body,
        grid=(RB,),
        in_specs=[
            pl.BlockSpec((BM, D), lambda i: (i, 0)),
            pl.BlockSpec((BM, D), lambda i: (i, 0)),
            pl.BlockSpec((D, D), lambda i: (0, 0)),
            pl.BlockSpec((NC, BM, D), lambda i: (0, i, 0)),
        ],
        out_specs=pl.BlockSpec((BM, D), lambda i: (i, 0)),
        out_shape=jax.ShapeDtypeStruct((N, D), jnp.float32),
    )(h, h0, M, agg2)


def _k4(h, W_final, b_final):
    return pl.pallas_call(
        _mm_body,
        grid=(RB,),
        in_specs=[
            pl.BlockSpec((BM, D), lambda i: (i, 0)),
            pl.BlockSpec((D, D), lambda i: (0, 0)),
            pl.BlockSpec((1, D), lambda i: (0, 0)),
        ],
        out_specs=pl.BlockSpec((BM, D), lambda i: (i, 0)),
        out_shape=jax.ShapeDtypeStruct((N, D), jnp.float32),
    )(h, W_final, b_final.reshape(1, D))


def kernel(x, edge_index, diffusion_distance, W_init, b_init,
           W_final, b_final):
    src_tiles = edge_index[0].reshape(NT, NCH, CH)
    dst_tiles = edge_index[1].reshape(NT, NCH, CH)
    dd_tiles = diffusion_distance.reshape(NT, NCH, CH)

    d0, d1 = _deg_partials(dst_tiles)
    h, w2 = _k1(x, W_init, b_init, d0, d1)
    w = w2.reshape(N)
    h0 = h
    for _ in range(NUM_LAYERS):
        M = _k2(h)
        agg2 = _edge_partials(h, w, src_tiles, dst_tiles, dd_tiles)
        h = _k3(h, h0, M, agg2)
    return _k4(h, W_final, b_final)


# speedup vs baseline: 5.2003x; 1.0271x over previous
import dataclasses

import jax
import jax.numpy as jnp
from jax import lax
from jax.experimental import pallas as pl
from jax.experimental.pallas import tpu as pltpu
from jax.experimental.pallas import tpu_sc as plsc

N = 10000
D = 128
E = 320000
NUM_LAYERS = 2
ALPHA = 0.1
BETA = 0.1
ETA = 0.5
EPS = 1e-5

NC = 2
NS = 16
NT = NC * NS
PER_TILE = E // NT
CH = 80
NCH = PER_TILE // CH
RPS = 624
TAIL = N - NS * RPS
RB = 5
BM = N // RB

_mesh = plsc.VectorSubcoreMesh(core_axis_name="c", subcore_axis_name="s")

_sc_params = pltpu.CompilerParams()
if "needs_layout_passes" in pltpu.CompilerParams.__dataclass_fields__:
    _sc_params = dataclasses.replace(_sc_params, needs_layout_passes=False)


def _deg_body(dst_hbm, out0_hbm, out1_hbm, deg_sh, dst_v, ones_v, zero_v):
    ci = lax.axis_index("c")
    si = lax.axis_index("s")
    tid = ci * NS + si

    @pl.loop(0, CH // 16)
    def _(i):
        ones_v[pl.ds(i * 16, 16)] = jnp.ones((16,), jnp.float32)

    @pl.when(si == 0)
    def _():
        @pl.loop(0, N // 16)
        def _(i):
            zero_v[pl.ds(i * 16, 16)] = jnp.zeros((16,), jnp.float32)
        pltpu.sync_copy(zero_v, deg_sh)

    pltpu.sync_copy(dst_hbm.at[tid], dst_v)
    plsc.subcore_barrier()

    @pl.loop(0, NCH)
    def _(c):
        pltpu.sync_copy(ones_v, deg_sh.at[dst_v.at[c]], add=True)

    plsc.subcore_barrier()

    @pl.when(jnp.logical_and(si == 0, ci == 0))
    def _():
        pltpu.sync_copy(deg_sh, out0_hbm)

    @pl.when(jnp.logical_and(si == 0, ci == 1))
    def _():
        pltpu.sync_copy(deg_sh, out1_hbm)


def _edge_body(h_hbm, w_hbm, src_hbm, dst_hbm, dd_hbm, out_hbm,
               agg_sh, w_v, sidx_v, didx_v, didx2_v, ddc_v, ws_v, wd_v,
               hs_v, hd_v, msg_v, sem1, sem2, sem3, sem4):
    ci = lax.axis_index("c")
    si = lax.axis_index("s")
    tid = ci * NS + si

    pltpu.sync_copy(w_hbm, w_v)

    @pl.loop(0, CH)
    def _(r):
        for cc in range(D // 16):
            msg_v[r, pl.ds(cc * 16, 16)] = jnp.zeros((16,), jnp.float32)

    for t in range(RPS // CH):
        pltpu.sync_copy(msg_v, agg_sh.at[pl.ds(si * RPS + t * CH, CH)])
    rem = RPS - (RPS // CH) * CH
    pltpu.sync_copy(msg_v.at[pl.ds(0, rem)],
                    agg_sh.at[pl.ds(si * RPS + RPS - rem, rem)])

    @pl.when(si == NS - 1)
    def _():
        pltpu.sync_copy(msg_v.at[pl.ds(0, TAIL)],
                        agg_sh.at[pl.ds(NS * RPS, TAIL)])

    plsc.subcore_barrier()

    def _do_chunk(c, hsbuf, dibuf, scsem):
        pltpu.sync_copy(src_hbm.at[tid, c], sidx_v)
        pltpu.sync_copy(dst_hbm.at[tid, c], dibuf)
        pltpu.sync_copy(dd_hbm.at[tid, c], ddc_v)
        g1 = pltpu.async_copy(h_hbm.at[sidx_v], hsbuf, sem1)
        g2 = pltpu.async_copy(h_hbm.at[dibuf], hd_v, sem2)
        for g in range(CH // 16):
            rb = g * 16
            src16 = sidx_v[pl.ds(rb, 16)]
            dst16 = dibuf[pl.ds(rb, 16)]
            ws_v[pl.ds(rb, 16)] = plsc.load_gather(w_v, [src16])
            wd_v[pl.ds(rb, 16)] = plsc.load_gather(w_v, [dst16])
        g1.wait()
        g2.wait()

        @pl.loop(0, CH)
        def _edge(e):
            ev = lax.broadcast(e, (16,))
            ws = plsc.load_gather(ws_v, [ev])
            wd = plsc.load_gather(wd_v, [ev])
            dd = plsc.load_gather(ddc_v, [ev])
            hs_c = [hsbuf[e, pl.ds(k * 16, 16)] for k in range(D // 16)]
            hd_c = [hd_v[e, pl.ds(k * 16, 16)] for k in range(D // 16)]
            acc = jnp.zeros((16,), jnp.float32)
            for k in range(D // 16):
                t = wd * hd_c[k] - ws * hs_c[k]
                acc = acc + t * t
            s = lax.broadcast(jnp.sum(acc), (16,))
            scl = jnp.maximum(s, 1e-20)
            ib = lax.bitcast_convert_type(scl, jnp.int32)
            ib = jnp.int32(0x5F3759DF) - lax.shift_right_arithmetic(ib, 1)
            y = lax.bitcast_convert_type(ib, jnp.float32)
            for _ in range(3):
                y = y * (1.5 - 0.5 * scl * y * y)
            denom = scl * y + EPS
            q = (ETA * dd) / denom
            a = ws * wd * (1.0 - q)
            b = (ws * ws) * q
            for k in range(D // 16):
                hsbuf[e, pl.ds(k * 16, 16)] = a * hs_c[k] + b * hd_c[k]

        return pltpu.async_copy(hsbuf, agg_sh.at[dibuf], scsem, add=True)

    @pl.loop(0, NCH // 2)
    def _pair(p):
        @pl.when(p > 0)
        def _():
            pltpu.make_async_copy(hs_v, agg_sh.at[didx_v], sem3).wait()

        s_even = _do_chunk(2 * p, hs_v, didx_v, sem3)

        @pl.when(p > 0)
        def _():
            pltpu.make_async_copy(msg_v, agg_sh.at[didx2_v], sem4).wait()

        s_odd = _do_chunk(2 * p + 1, msg_v, didx2_v, sem4)

        @pl.when(p == NCH // 2 - 1)
        def _():
            s_even.wait()
            _do_chunk(2 * p + 2, hs_v, didx_v, sem3).wait()
            s_odd.wait()

    plsc.subcore_barrier()
    pltpu.sync_copy(agg_sh.at[pl.ds(si * RPS, RPS)],
                    out_hbm.at[ci, pl.ds(si * RPS, RPS)])

    @pl.when(si == NS - 1)
    def _():
        pltpu.sync_copy(agg_sh.at[pl.ds(NS * RPS, TAIL)],
                        out_hbm.at[ci, pl.ds(NS * RPS, TAIL)])


def _deg_partials(dst_tiles):
    return pl.kernel(
        _deg_body,
        out_type=[jax.ShapeDtypeStruct((N,), jnp.float32),
                  jax.ShapeDtypeStruct((N,), jnp.float32)],
        mesh=_mesh,
        compiler_params=_sc_params,
        scratch_types=[
            pltpu.VMEM_SHARED((N,), jnp.float32),
            pltpu.VMEM((NCH, CH), jnp.int32),
            pltpu.VMEM((CH,), jnp.float32),
            pltpu.VMEM((N,), jnp.float32),
        ],
    )(dst_tiles)


def _edge_partials(h, w, src_tiles, dst_tiles, dd_tiles):
    return pl.kernel(
        _edge_body,
        out_type=jax.ShapeDtypeStruct((NC, N, D), jnp.float32),
        mesh=_mesh,
        compiler_params=_sc_params,
        scratch_types=[
            pltpu.VMEM_SHARED((N, D), jnp.float32),
            pltpu.VMEM((N,), jnp.float32),
            pltpu.VMEM((CH,), jnp.int32),
            pltpu.VMEM((CH,), jnp.int32),
            pltpu.VMEM((CH,), jnp.int32),
            pltpu.VMEM((CH,), jnp.float32),
            pltpu.VMEM((CH,), jnp.float32),
            pltpu.VMEM((CH,), jnp.float32),
            pltpu.VMEM((CH, D), jnp.float32),
            pltpu.VMEM((CH, D), jnp.float32),
            pltpu.VMEM((CH, D), jnp.float32),
            pltpu.SemaphoreType.DMA,
            pltpu.SemaphoreType.DMA,
            pltpu.SemaphoreType.DMA,
            pltpu.SemaphoreType.DMA,
        ],
    )(h, w, src_tiles, dst_tiles, dd_tiles)


def _k1_body(x_ref, wi_ref, bi_ref, d0_ref, d1_ref, h_ref, w_ref):
    h_ref[...] = (jnp.dot(x_ref[...], wi_ref[...],
                          preferred_element_type=jnp.float32) + bi_ref[...])
    deg = d0_ref[...] + d1_ref[...]
    w_ref[...] = jnp.where(deg > 0.0, lax.rsqrt(jnp.maximum(deg, 1.0)), 0.0)


def _k2_body(h_ref, m_ref, acc_ref):
    i = pl.program_id(0)

    @pl.when(i == 0)
    def _():
        acc_ref[...] = jnp.zeros_like(acc_ref)

    hb = h_ref[...]
    acc_ref[...] += lax.dot_general(hb, hb, (((0,), (0,)), ((), ())),
                                    preferred_element_type=jnp.float32)

    @pl.when(i == RB - 1)
    def _():
        G = acc_ref[...]
        rr = lax.broadcasted_iota(jnp.int32, (D, D), 0)
        cc = lax.broadcasted_iota(jnp.int32, (D, D), 1)
        diag = jnp.where(rr == cc, G, 0.0)
        drow = jnp.sum(diag, axis=1, keepdims=True)
        dcol = jnp.sum(diag, axis=0, keepdims=True)
        crow = 1.0 / jnp.maximum(jnp.sqrt(drow), 1e-12)
        ccol = 1.0 / jnp.maximum(jnp.sqrt(dcol), 1e-12)
        m_ref[...] = (crow * crow) * G * ccol


def _k3_body(h_ref, h0_ref, m_ref, agg_ref, o_ref):
    o_ref[...] = jnp.maximum(
        ALPHA * h0_ref[...]
        - BETA * jnp.dot(h_ref[...], m_ref[...],
                         preferred_element_type=jnp.float32)
        + agg_ref[0] + agg_ref[1],
        0.0)


def _mm_body(h_ref, wf_ref, bf_ref, o_ref):
    o_ref[...] = (jnp.dot(h_ref[...], wf_ref[...],
                          preferred_element_type=jnp.float32) + bf_ref[...])


def _k1(x, W_init, b_init, d0, d1):
    return pl.pallas_call(
        _k1_body,
        grid=(RB,),
        in_specs=[
            pl.BlockSpec((BM, D), lambda i: (i, 0)),
            pl.BlockSpec((D, D), lambda i: (0, 0)),
            pl.BlockSpec((1, D), lambda i: (0, 0)),
            pl.BlockSpec((RB, BM), lambda i: (0, 0)),
            pl.BlockSpec((RB, BM), lambda i: (0, 0)),
        ],
        out_specs=[
            pl.BlockSpec((BM, D), lambda i: (i, 0)),
            pl.BlockSpec((RB, BM), lambda i: (0, 0)),
        ],
        out_shape=[
            jax.ShapeDtypeStruct((N, D), jnp.float32),
            jax.ShapeDtypeStruct((RB, BM), jnp.float32),
        ],
    )(x, W_init, b_init.reshape(1, D), d0.reshape(RB, BM), d1.reshape(RB, BM))


def _k2(h):
    return pl.pallas_call(
        _k2_body,
        grid=(RB,),
        in_specs=[pl.BlockSpec((BM, D), lambda i: (i, 0))],
        out_specs=pl.BlockSpec((D, D), lambda i: (0, 0)),
        out_shape=jax.ShapeDtypeStruct((D, D), jnp.float32),
        scratch_shapes=[pltpu.VMEM((D, D), jnp.float32)],
    )(h)


def _k3(h, h0, M, agg2):
    return pl.pallas_call(
        _k3_body,
        grid=(RB,),
        in_specs=[
            pl.BlockSpec((BM, D), lambda i: (i, 0)),
            pl.BlockSpec((BM, D), lambda i: (i, 0)),
            pl.BlockSpec((D, D), lambda i: (0, 0)),
            pl.BlockSpec((NC, BM, D), lambda i: (0, i, 0)),
        ],
        out_specs=pl.BlockSpec((BM, D), lambda i: (i, 0)),
        out_shape=jax.ShapeDtypeStruct((N, D), jnp.float32),
    )(h, h0, M, agg2)


def _k4(h, W_final, b_final):
    return pl.pallas_call(
        _mm_body,
        grid=(RB,),
        in_specs=[
            pl.BlockSpec((BM, D), lambda i: (i, 0)),
            pl.BlockSpec((D, D), lambda i: (0, 0)),
            pl.BlockSpec((1, D), lambda i: (0, 0)),
        ],
        out_specs=pl.BlockSpec((BM, D), lambda i: (i, 0)),
        out_shape=jax.ShapeDtypeStruct((N, D), jnp.float32),
    )(h, W_final, b_final.reshape(1, D))


def kernel(x, edge_index, diffusion_distance, W_init, b_init,
           W_final, b_final):
    src_tiles = edge_index[0].reshape(NT, NCH, CH)
    dst_tiles = edge_index[1].reshape(NT, NCH, CH)
    dd_tiles = diffusion_distance.reshape(NT, NCH, CH)

    d0, d1 = _deg_partials(dst_tiles)
    h, w2 = _k1(x, W_init, b_init, d0, d1)
    w = w2.reshape(N)
    h0 = h
    for _ in range(NUM_LAYERS):
        M = _k2(h)
        agg2 = _edge_partials(h, w, src_tiles, dst_tiles, dd_tiles)
        h = _k3(h, h0, M, agg2)
    return _k4(h, W_final, b_final)


# speedup vs baseline: 5.8439x; 1.1237x over previous
import dataclasses

import jax
import jax.numpy as jnp
from jax import lax
from jax.experimental import pallas as pl
from jax.experimental.pallas import tpu as pltpu
from jax.experimental.pallas import tpu_sc as plsc

N = 10000
D = 128
E = 320000
NUM_LAYERS = 2
ALPHA = 0.1
BETA = 0.1
ETA = 0.5
EPS = 1e-5

NC = 2
NS = 16
NT = NC * NS
PER_TILE = E // NT
CH = 80
NCH = PER_TILE // CH
RPS = 624
TAIL = N - NS * RPS
RB = 5
BM = N // RB

_mesh = plsc.VectorSubcoreMesh(core_axis_name="c", subcore_axis_name="s")

_sc_params = pltpu.CompilerParams()
if "needs_layout_passes" in pltpu.CompilerParams.__dataclass_fields__:
    _sc_params = dataclasses.replace(_sc_params, needs_layout_passes=False)


def _deg_body(dst_hbm, out0_hbm, out1_hbm, deg_sh, dst_v, ones_v, zero_v):
    ci = lax.axis_index("c")
    si = lax.axis_index("s")
    tid = ci * NS + si

    @pl.loop(0, CH // 16)
    def _(i):
        ones_v[pl.ds(i * 16, 16)] = jnp.ones((16,), jnp.float32)

    @pl.when(si == 0)
    def _():
        @pl.loop(0, N // 16)
        def _(i):
            zero_v[pl.ds(i * 16, 16)] = jnp.zeros((16,), jnp.float32)
        pltpu.sync_copy(zero_v, deg_sh)

    pltpu.sync_copy(dst_hbm.at[tid], dst_v)
    plsc.subcore_barrier()

    @pl.loop(0, NCH)
    def _(c):
        pltpu.sync_copy(ones_v, deg_sh.at[dst_v.at[c]], add=True)

    plsc.subcore_barrier()

    @pl.when(jnp.logical_and(si == 0, ci == 0))
    def _():
        pltpu.sync_copy(deg_sh, out0_hbm)

    @pl.when(jnp.logical_and(si == 0, ci == 1))
    def _():
        pltpu.sync_copy(deg_sh, out1_hbm)


def _edge_body(h_hbm, w_hbm, src_hbm, dst_hbm, dd_hbm, out_hbm,
               agg_sh, w_v, sidx_v, didx_v, didx2_v, ddc_v, ws_v, wd_v,
               hs_v, hd_v, msg_v, sem1, sem2, sem3, sem4, sem5, sem6, sem7):
    ci = lax.axis_index("c")
    si = lax.axis_index("s")
    tid = ci * NS + si

    pltpu.sync_copy(w_hbm, w_v)

    @pl.loop(0, CH)
    def _(r):
        for cc in range(D // 16):
            msg_v[r, pl.ds(cc * 16, 16)] = jnp.zeros((16,), jnp.float32)

    for t in range(RPS // CH):
        pltpu.sync_copy(msg_v, agg_sh.at[pl.ds(si * RPS + t * CH, CH)])
    rem = RPS - (RPS // CH) * CH
    pltpu.sync_copy(msg_v.at[pl.ds(0, rem)],
                    agg_sh.at[pl.ds(si * RPS + RPS - rem, rem)])

    @pl.when(si == NS - 1)
    def _():
        pltpu.sync_copy(msg_v.at[pl.ds(0, TAIL)],
                        agg_sh.at[pl.ds(NS * RPS, TAIL)])

    plsc.subcore_barrier()

    def _do_chunk(c, hsbuf, dibuf, scsem):
        i1 = pltpu.async_copy(src_hbm.at[tid, c], sidx_v, sem5)
        i2 = pltpu.async_copy(dst_hbm.at[tid, c], dibuf, sem6)
        i3 = pltpu.async_copy(dd_hbm.at[tid, c], ddc_v, sem7)
        i1.wait()
        i2.wait()
        g1 = pltpu.async_copy(h_hbm.at[sidx_v], hsbuf, sem1)
        g2 = pltpu.async_copy(h_hbm.at[dibuf], hd_v, sem2)
        i3.wait()
        for g in range(CH // 16):
            rb = g * 16
            src16 = sidx_v[pl.ds(rb, 16)]
            dst16 = dibuf[pl.ds(rb, 16)]
            ws_v[pl.ds(rb, 16)] = plsc.load_gather(w_v, [src16])
            wd_v[pl.ds(rb, 16)] = plsc.load_gather(w_v, [dst16])
        g1.wait()
        g2.wait()

        @pl.loop(0, CH)
        def _edge(e):
            ev = lax.broadcast(e, (16,))
            ws = plsc.load_gather(ws_v, [ev])
            wd = plsc.load_gather(wd_v, [ev])
            dd = plsc.load_gather(ddc_v, [ev])
            hs_c = [hsbuf[e, pl.ds(k * 16, 16)] for k in range(D // 16)]
            hd_c = [hd_v[e, pl.ds(k * 16, 16)] for k in range(D // 16)]
            acc = jnp.zeros((16,), jnp.float32)
            for k in range(D // 16):
                t = wd * hd_c[k] - ws * hs_c[k]
                acc = acc + t * t
            s = lax.broadcast(jnp.sum(acc), (16,))
            scl = jnp.maximum(s, 1e-20)
            ib = lax.bitcast_convert_type(scl, jnp.int32)
            ib = jnp.int32(0x5F3759DF) - lax.shift_right_arithmetic(ib, 1)
            y = lax.bitcast_convert_type(ib, jnp.float32)
            for _ in range(3):
                y = y * (1.5 - 0.5 * scl * y * y)
            denom = scl * y + EPS
            q = (ETA * dd) / denom
            a = ws * wd * (1.0 - q)
            b = (ws * ws) * q
            for k in range(D // 16):
                hsbuf[e, pl.ds(k * 16, 16)] = a * hs_c[k] + b * hd_c[k]

        return pltpu.async_copy(hsbuf, agg_sh.at[dibuf], scsem, add=True)

    @pl.loop(0, NCH // 2)
    def _pair(p):
        @pl.when(p > 0)
        def _():
            pltpu.make_async_copy(hs_v, agg_sh.at[didx_v], sem3).wait()

        s_even = _do_chunk(2 * p, hs_v, didx_v, sem3)

        @pl.when(p > 0)
        def _():
            pltpu.make_async_copy(msg_v, agg_sh.at[didx2_v], sem4).wait()

        s_odd = _do_chunk(2 * p + 1, msg_v, didx2_v, sem4)

        @pl.when(p == NCH // 2 - 1)
        def _():
            s_even.wait()
            _do_chunk(2 * p + 2, hs_v, didx_v, sem3).wait()
            s_odd.wait()

    plsc.subcore_barrier()
    pltpu.sync_copy(agg_sh.at[pl.ds(si * RPS, RPS)],
                    out_hbm.at[ci, pl.ds(si * RPS, RPS)])

    @pl.when(si == NS - 1)
    def _():
        pltpu.sync_copy(agg_sh.at[pl.ds(NS * RPS, TAIL)],
                        out_hbm.at[ci, pl.ds(NS * RPS, TAIL)])


def _deg_partials(dst_tiles):
    return pl.kernel(
        _deg_body,
        out_type=[jax.ShapeDtypeStruct((N,), jnp.float32),
                  jax.ShapeDtypeStruct((N,), jnp.float32)],
        mesh=_mesh,
        compiler_params=_sc_params,
        scratch_types=[
            pltpu.VMEM_SHARED((N,), jnp.float32),
            pltpu.VMEM((NCH, CH), jnp.int32),
            pltpu.VMEM((CH,), jnp.float32),
            pltpu.VMEM((N,), jnp.float32),
        ],
    )(dst_tiles)


def _edge_partials(h, w, src_tiles, dst_tiles, dd_tiles):
    return pl.kernel(
        _edge_body,
        out_type=jax.ShapeDtypeStruct((NC, N, D), jnp.float32),
        mesh=_mesh,
        compiler_params=_sc_params,
        scratch_types=[
            pltpu.VMEM_SHARED((N, D), jnp.float32),
            pltpu.VMEM((N,), jnp.float32),
            pltpu.VMEM((CH,), jnp.int32),
            pltpu.VMEM((CH,), jnp.int32),
            pltpu.VMEM((CH,), jnp.int32),
            pltpu.VMEM((CH,), jnp.float32),
            pltpu.VMEM((CH,), jnp.float32),
            pltpu.VMEM((CH,), jnp.float32),
            pltpu.VMEM((CH, D), jnp.float32),
            pltpu.VMEM((CH, D), jnp.float32),
            pltpu.VMEM((CH, D), jnp.float32),
            pltpu.SemaphoreType.DMA,
            pltpu.SemaphoreType.DMA,
            pltpu.SemaphoreType.DMA,
            pltpu.SemaphoreType.DMA,
            pltpu.SemaphoreType.DMA,
            pltpu.SemaphoreType.DMA,
            pltpu.SemaphoreType.DMA,
        ],
    )(h, w, src_tiles, dst_tiles, dd_tiles)


def _k1_body(x_ref, wi_ref, bi_ref, d0_ref, d1_ref, h_ref, w_ref):
    h_ref[...] = (jnp.dot(x_ref[...], wi_ref[...],
                          preferred_element_type=jnp.float32) + bi_ref[...])
    deg = d0_ref[...] + d1_ref[...]
    w_ref[...] = jnp.where(deg > 0.0, lax.rsqrt(jnp.maximum(deg, 1.0)), 0.0)


def _k2_body(h_ref, m_ref, acc_ref):
    i = pl.program_id(0)

    @pl.when(i == 0)
    def _():
        acc_ref[...] = jnp.zeros_like(acc_ref)

    hb = h_ref[...]
    acc_ref[...] += lax.dot_general(hb, hb, (((0,), (0,)), ((), ())),
                                    preferred_element_type=jnp.float32)

    @pl.when(i == RB - 1)
    def _():
        G = acc_ref[...]
        rr = lax.broadcasted_iota(jnp.int32, (D, D), 0)
        cc = lax.broadcasted_iota(jnp.int32, (D, D), 1)
        diag = jnp.where(rr == cc, G, 0.0)
        drow = jnp.sum(diag, axis=1, keepdims=True)
        dcol = jnp.sum(diag, axis=0, keepdims=True)
        crow = 1.0 / jnp.maximum(jnp.sqrt(drow), 1e-12)
        ccol = 1.0 / jnp.maximum(jnp.sqrt(dcol), 1e-12)
        m_ref[...] = (crow * crow) * G * ccol


def _k3_body(h_ref, h0_ref, m_ref, agg_ref, o_ref):
    o_ref[...] = jnp.maximum(
        ALPHA * h0_ref[...]
        - BETA * jnp.dot(h_ref[...], m_ref[...],
                         preferred_element_type=jnp.float32)
        + agg_ref[0] + agg_ref[1],
        0.0)


def _mm_body(h_ref, wf_ref, bf_ref, o_ref):
    o_ref[...] = (jnp.dot(h_ref[...], wf_ref[...],
                          preferred_element_type=jnp.float32) + bf_ref[...])


def _k1(x, W_init, b_init, d0, d1):
    return pl.pallas_call(
        _k1_body,
        grid=(RB,),
        in_specs=[
            pl.BlockSpec((BM, D), lambda i: (i, 0)),
            pl.BlockSpec((D, D), lambda i: (0, 0)),
            pl.BlockSpec((1, D), lambda i: (0, 0)),
            pl.BlockSpec((RB, BM), lambda i: (0, 0)),
            pl.BlockSpec((RB, BM), lambda i: (0, 0)),
        ],
        out_specs=[
            pl.BlockSpec((BM, D), lambda i: (i, 0)),
            pl.BlockSpec((RB, BM), lambda i: (0, 0)),
        ],
        out_shape=[
            jax.ShapeDtypeStruct((N, D), jnp.float32),
            jax.ShapeDtypeStruct((RB, BM), jnp.float32),
        ],
    )(x, W_init, b_init.reshape(1, D), d0.reshape(RB, BM), d1.reshape(RB, BM))


def _k2(h):
    return pl.pallas_call(
        _k2_body,
        grid=(RB,),
        in_specs=[pl.BlockSpec((BM, D), lambda i: (i, 0))],
        out_specs=pl.BlockSpec((D, D), lambda i: (0, 0)),
        out_shape=jax.ShapeDtypeStruct((D, D), jnp.float32),
        scratch_shapes=[pltpu.VMEM((D, D), jnp.float32)],
    )(h)


def _k3(h, h0, M, agg2):
    return pl.pallas_call(
        _k3_body,
        grid=(RB,),
        in_specs=[
            pl.BlockSpec((BM, D), lambda i: (i, 0)),
            pl.BlockSpec((BM, D), lambda i: (i, 0)),
            pl.BlockSpec((D, D), lambda i: (0, 0)),
            pl.BlockSpec((NC, BM, D), lambda i: (0, i, 0)),
        ],
        out_specs=pl.BlockSpec((BM, D), lambda i: (i, 0)),
        out_shape=jax.ShapeDtypeStruct((N, D), jnp.float32),
    )(h, h0, M, agg2)


def _k4(h, W_final, b_final):
    return pl.pallas_call(
        _mm_body,
        grid=(RB,),
        in_specs=[
            pl.BlockSpec((BM, D), lambda i: (i, 0)),
            pl.BlockSpec((D, D), lambda i: (0, 0)),
            pl.BlockSpec((1, D), lambda i: (0, 0)),
        ],
        out_specs=pl.BlockSpec((BM, D), lambda i: (i, 0)),
        out_shape=jax.ShapeDtypeStruct((N, D), jnp.float32),
    )(h, W_final, b_final.reshape(1, D))


def kernel(x, edge_index, diffusion_distance, W_init, b_init,
           W_final, b_final):
    src_tiles = edge_index[0].reshape(NT, NCH, CH)
    dst_tiles = edge_index[1].reshape(NT, NCH, CH)
    dd_tiles = diffusion_distance.reshape(NT, NCH, CH)

    d0, d1 = _deg_partials(dst_tiles)
    h, w2 = _k1(x, W_init, b_init, d0, d1)
    w = w2.reshape(N)
    h0 = h
    for _ in range(NUM_LAYERS):
        M = _k2(h)
        agg2 = _edge_partials(h, w, src_tiles, dst_tiles, dd_tiles)
        h = _k3(h, h0, M, agg2)
    return _k4(h, W_final, b_final)
